# baseline (speedup 1.0000x reference)
import jax
import jax.numpy as jnp
from jax.experimental import pallas as pl
from jax.experimental.pallas import tpu as pltpu

NBUF = 4
CH = 128


def _add_body(x_hbm, pe_ref, o_hbm, xbuf, obuf, in_sems, out_sems):
    nchunk = x_hbm.shape[0] // CH

    def in_copy(i, slot):
        return pltpu.make_async_copy(
            x_hbm.at[pl.ds(i * CH, CH)], xbuf.at[slot], in_sems.at[slot]
        )

    def out_copy(i, slot):
        return pltpu.make_async_copy(
            obuf.at[slot], o_hbm.at[pl.ds(i * CH, CH)], out_sems.at[slot]
        )

    for k in range(NBUF):
        in_copy(k, k).start()

    def loop(i, carry):
        slot = jax.lax.rem(i, NBUF)
        in_copy(i, slot).wait()

        @pl.when(i >= NBUF)
        def _():
            out_copy(i - NBUF, slot).wait()

        obuf[slot] = xbuf[slot] + pe_ref[...]
        out_copy(i, slot).start()

        @pl.when(i + NBUF < nchunk)
        def _():
            in_copy(i + NBUF, slot).start()

        return carry

    jax.lax.fori_loop(0, nchunk, loop, 0)

    for k in range(NBUF):
        i = nchunk - NBUF + k
        out_copy(i, i % NBUF).wait()


def kernel(x, pos_emb):
    B, L, D = x.shape
    LD = L * D
    x2 = x.reshape(B, LD)
    pe2 = pos_emb[:L].reshape(1, LD)
    out = pl.pallas_call(
        _add_body,
        in_specs=[
            pl.BlockSpec(memory_space=pl.ANY),
            pl.BlockSpec(memory_space=pltpu.MemorySpace.VMEM),
        ],
        out_specs=pl.BlockSpec(memory_space=pl.ANY),
        out_shape=jax.ShapeDtypeStruct((B, LD), x.dtype),
        scratch_shapes=[
            pltpu.MemorySpace.VMEM((NBUF, CH, LD), jnp.float32),
            pltpu.MemorySpace.VMEM((NBUF, CH, LD), jnp.float32),
            pltpu.SemaphoreType.DMA((NBUF,)),
            pltpu.SemaphoreType.DMA((NBUF,)),
        ],
    )(x2, pe2)
    return out.reshape(B, L, D)
